# baseline (speedup 1.0000x reference)
import functools

import jax
import jax.numpy as jnp
from jax import lax
from jax.experimental import pallas as pl
from jax.experimental.pallas import tpu as pltpu
from jax.experimental.pallas import tpu_sc as plsc

NUM_CORES = 2
NUM_SUBCORES = 16
NUM_WORKERS = NUM_CORES * NUM_SUBCORES

BATCH = 4096
SEQ = 200
EMB = 60
VOCAB = 1000000
PAIR = 2 * EMB
VOCAB_PAIRS = VOCAB // 2
TOTAL = BATCH * SEQ
PER_WORKER = TOTAL // NUM_WORKERS
CHUNK = 128
CHUNKS = PER_WORKER // CHUNK
NBUF = 4


def _gather_body(pair_id_hbm, soff_hbm, pairs_hbm, out_hbm,
                 pid_v, soff_v, pair_v, packed_v,
                 sem_i, sem_s, sem_g, sem_o):
    wid = lax.axis_index("s") * NUM_CORES + lax.axis_index("c")
    base = wid * PER_WORKER

    def issue_idx(c, b):
        pltpu.async_copy(
            pair_id_hbm.at[pl.ds(base + c * CHUNK, CHUNK)],
            pid_v.at[b], sem_i.at[b],
        )
        pltpu.async_copy(
            soff_hbm.at[pl.ds(base + c * CHUNK, CHUNK)],
            soff_v.at[b], sem_s.at[b],
        )

    def wait_idx(b):
        pltpu.make_async_copy(
            pair_id_hbm.at[pl.ds(0, CHUNK)], pid_v.at[b], sem_i.at[b]
        ).wait()
        pltpu.make_async_copy(
            soff_hbm.at[pl.ds(0, CHUNK)], soff_v.at[b], sem_s.at[b]
        ).wait()

    def issue_gather(b):
        pltpu.async_copy(
            pairs_hbm.at[pid_v.at[b]], pair_v.at[b], sem_g.at[b]
        )

    def wait_gather(b):
        pltpu.make_async_copy(
            pairs_hbm.at[pl.ds(0, CHUNK)], pair_v.at[b], sem_g.at[b]
        ).wait()

    def issue_out(c, b):
        pltpu.async_copy(
            packed_v.at[b],
            out_hbm.at[pl.ds(EMB * (base + c * CHUNK), EMB * CHUNK)],
            sem_o.at[b],
        )

    def wait_out(b):
        pltpu.make_async_copy(
            packed_v.at[b], out_hbm.at[pl.ds(0, EMB * CHUNK)], sem_o.at[b]
        ).wait()

    for b in range(NBUF):
        issue_idx(b, b)
    for b in range(NBUF):
        wait_idx(b)
        issue_gather(b)

    def super_step(cc, carry):
        for b in range(NBUF):
            c = NBUF * cc + b
            wait_gather(b)

            @pl.when(cc > 0)
            def _():
                wait_out(b)

            def group(g, carry2):
                svec = soff_v[b, pl.ds(g * 16, 16)]
                for j in range(16):
                    s = svec[j]
                    k = g * 16 + j
                    src = pair_v.at[b, k]
                    d = EMB * k
                    for m in (0, 16, 32, 44):
                        packed_v[b, pl.ds(d + m, 16)] = src[pl.ds(s + m, 16)]
                return carry2

            lax.fori_loop(0, CHUNK // 16, group, 0)
            issue_out(c, b)

            @pl.when(cc < CHUNKS // NBUF - 1)
            def _():
                issue_idx(c + NBUF, b)
                wait_idx(b)
                issue_gather(b)
        return carry

    lax.fori_loop(0, CHUNKS // NBUF, super_step, 0)
    for b in range(NBUF):
        wait_out(b)


@jax.jit
def _embedding_gather(pair_id, soff, pairs):
    mesh = plsc.VectorSubcoreMesh(
        core_axis_name="c", subcore_axis_name="s",
        num_cores=NUM_CORES, num_subcores=NUM_SUBCORES,
    )
    run = pl.kernel(
        _gather_body,
        out_type=jax.ShapeDtypeStruct((TOTAL * EMB,), jnp.float32),
        mesh=mesh,
        scratch_types=[
            pltpu.VMEM((NBUF, CHUNK), jnp.int32),
            pltpu.VMEM((NBUF, CHUNK), jnp.int32),
            pltpu.VMEM((NBUF, CHUNK, PAIR), jnp.float32),
            pltpu.VMEM((NBUF, CHUNK * EMB), jnp.float32),
            pltpu.SemaphoreType.DMA((NBUF,)),
            pltpu.SemaphoreType.DMA((NBUF,)),
            pltpu.SemaphoreType.DMA((NBUF,)),
            pltpu.SemaphoreType.DMA((NBUF,)),
        ],
        compiler_params=pltpu.CompilerParams(use_tc_tiling_on_sc=False),
    )
    return run(pair_id, soff, pairs)


def kernel(unit_id_seqs, unit_embedding_table):
    idx_flat = unit_id_seqs.reshape(TOTAL).astype(jnp.int32)
    pair_id = idx_flat >> 1
    soff = (idx_flat & 1) * EMB
    pairs = unit_embedding_table.reshape(VOCAB_PAIRS, PAIR)
    out = _embedding_gather(pair_id, soff, pairs)
    return out.reshape(BATCH, SEQ, EMB)


# speedup vs baseline: 1.0396x; 1.0396x over previous
import functools

import jax
import jax.numpy as jnp
from jax import lax
from jax.experimental import pallas as pl
from jax.experimental.pallas import tpu as pltpu
from jax.experimental.pallas import tpu_sc as plsc

NUM_CORES = 2
NUM_SUBCORES = 16
NUM_WORKERS = NUM_CORES * NUM_SUBCORES

BATCH = 4096
SEQ = 200
EMB = 60
VOCAB = 1000000
PAIR = 2 * EMB
VOCAB_PAIRS = VOCAB // 2
TOTAL = BATCH * SEQ
PER_WORKER = TOTAL // NUM_WORKERS
CHUNK = 128
CHUNKS = PER_WORKER // CHUNK
NBUF = 4


def _gather_body(idx_hbm, pairs_hbm, out_hbm,
                 idx_v, pid_v, pair_v, packed_v, sem_g, sem_o):
    wid = lax.axis_index("s") * NUM_CORES + lax.axis_index("c")
    base = wid * PER_WORKER

    pltpu.sync_copy(idx_hbm.at[pl.ds(base, PER_WORKER)], idx_v)

    def fill_and_gather(c, b):
        for j in range(CHUNK // 16):
            pid_v[b, pl.ds(j * 16, 16)] = (
                idx_v[pl.ds(c * CHUNK + j * 16, 16)] >> 1
            )
        pltpu.async_copy(
            pairs_hbm.at[pid_v.at[b]], pair_v.at[b], sem_g.at[b]
        )

    def wait_gather(b):
        pltpu.make_async_copy(
            pairs_hbm.at[pl.ds(0, CHUNK)], pair_v.at[b], sem_g.at[b]
        ).wait()

    def issue_out(c, b):
        pltpu.async_copy(
            packed_v.at[b],
            out_hbm.at[pl.ds(EMB * (base + c * CHUNK), EMB * CHUNK)],
            sem_o.at[b],
        )

    def wait_out(b):
        pltpu.make_async_copy(
            packed_v.at[b], out_hbm.at[pl.ds(0, EMB * CHUNK)], sem_o.at[b]
        ).wait()

    for b in range(NBUF):
        fill_and_gather(b, b)

    def super_step(cc, carry):
        for b in range(NBUF):
            c = NBUF * cc + b
            wait_gather(b)

            @pl.when(cc > 0)
            def _():
                wait_out(b)

            def group(g, carry2):
                svec = (idx_v[pl.ds(c * CHUNK + g * 16, 16)] & 1) * EMB
                for j in range(16):
                    s = svec[j]
                    k = g * 16 + j
                    src = pair_v.at[b, k]
                    d = EMB * k
                    for m in (0, 16, 32, 44):
                        packed_v[b, pl.ds(d + m, 16)] = src[pl.ds(s + m, 16)]
                return carry2

            lax.fori_loop(0, CHUNK // 16, group, 0)
            issue_out(c, b)

            @pl.when(cc < CHUNKS // NBUF - 1)
            def _():
                fill_and_gather(c + NBUF, b)
        return carry

    lax.fori_loop(0, CHUNKS // NBUF, super_step, 0)
    for b in range(NBUF):
        wait_out(b)


@jax.jit
def _embedding_gather(idx_flat, pairs):
    mesh = plsc.VectorSubcoreMesh(
        core_axis_name="c", subcore_axis_name="s",
        num_cores=NUM_CORES, num_subcores=NUM_SUBCORES,
    )
    run = pl.kernel(
        _gather_body,
        out_type=jax.ShapeDtypeStruct((TOTAL * EMB,), jnp.float32),
        mesh=mesh,
        scratch_types=[
            pltpu.VMEM((PER_WORKER,), jnp.int32),
            pltpu.VMEM((NBUF, CHUNK), jnp.int32),
            pltpu.VMEM((NBUF, CHUNK, PAIR), jnp.float32),
            pltpu.VMEM((NBUF, CHUNK * EMB), jnp.float32),
            pltpu.SemaphoreType.DMA((NBUF,)),
            pltpu.SemaphoreType.DMA((NBUF,)),
        ],
        compiler_params=pltpu.CompilerParams(use_tc_tiling_on_sc=False),
    )
    return run(idx_flat, pairs)


def kernel(unit_id_seqs, unit_embedding_table):
    idx_flat = unit_id_seqs.reshape(TOTAL).astype(jnp.int32)
    pairs = unit_embedding_table.reshape(VOCAB_PAIRS, PAIR)
    out = _embedding_gather(idx_flat, pairs)
    return out.reshape(BATCH, SEQ, EMB)


# speedup vs baseline: 1.0529x; 1.0128x over previous
import functools

import jax
import jax.numpy as jnp
from jax import lax
from jax.experimental import pallas as pl
from jax.experimental.pallas import tpu as pltpu
from jax.experimental.pallas import tpu_sc as plsc

NUM_CORES = 2
NUM_SUBCORES = 16
NUM_WORKERS = NUM_CORES * NUM_SUBCORES

BATCH = 4096
SEQ = 200
EMB = 60
VOCAB = 1000000
PAIR = 2 * EMB
VOCAB_PAIRS = VOCAB // 2
TOTAL = BATCH * SEQ
PER_WORKER = TOTAL // NUM_WORKERS
CHUNK = 128
CHUNKS = PER_WORKER // CHUNK


def _gather_body(idx_hbm, pair_id_hbm, pairs_hbm, out_hbm,
                 idx_v, pid_v, pair_v, packed_v, sem_g, sem_o):
    wid = lax.axis_index("s") * NUM_CORES + lax.axis_index("c")
    base = wid * PER_WORKER

    pltpu.sync_copy(idx_hbm.at[pl.ds(base, PER_WORKER)], idx_v)
    pltpu.sync_copy(pair_id_hbm.at[pl.ds(base, PER_WORKER)], pid_v)

    def issue_gather(c, b):
        pltpu.async_copy(
            pairs_hbm.at[pid_v.at[pl.ds(c * CHUNK, CHUNK)]],
            pair_v.at[b], sem_g.at[b],
        )

    def wait_gather(b):
        pltpu.make_async_copy(
            pairs_hbm.at[pl.ds(0, CHUNK)], pair_v.at[b], sem_g.at[b]
        ).wait()

    def issue_out(c, b):
        pltpu.async_copy(
            packed_v.at[b],
            out_hbm.at[pl.ds(EMB * (base + c * CHUNK), EMB * CHUNK)],
            sem_o.at[b],
        )

    def wait_out(b):
        pltpu.make_async_copy(
            packed_v.at[b], out_hbm.at[pl.ds(0, EMB * CHUNK)], sem_o.at[b]
        ).wait()

    NBUF = 3
    for b in range(NBUF):
        issue_gather(b, b)

    def super_step(cc, carry):
        for b in range(NBUF):
            c = NBUF * cc + b
            wait_gather(b)

            @pl.when(cc > 0)
            def _():
                wait_out(b)

            def group(g, carry2):
                par_vec = (idx_v[pl.ds(c * CHUNK + g * 16, 16)] & 1) * EMB
                for j in range(16):
                    s = par_vec[j]
                    k = g * 16 + j
                    src = pair_v.at[b, k]
                    d = EMB * k
                    for m in (0, 16, 32, 44):
                        packed_v[b, pl.ds(d + m, 16)] = src[pl.ds(s + m, 16)]
                return carry2

            lax.fori_loop(0, CHUNK // 16, group, 0)
            issue_out(c, b)

            @pl.when(cc < CHUNKS // NBUF - 1)
            def _():
                issue_gather(c + NBUF, b)
        return carry

    lax.fori_loop(0, CHUNKS // NBUF, super_step, 0)
    for b in range(NBUF):
        wait_out(b)


@jax.jit
def _embedding_gather(idx_flat, pair_id, pairs):
    mesh = plsc.VectorSubcoreMesh(
        core_axis_name="c", subcore_axis_name="s",
        num_cores=NUM_CORES, num_subcores=NUM_SUBCORES,
    )
    run = pl.kernel(
        _gather_body,
        out_type=jax.ShapeDtypeStruct((TOTAL * EMB,), jnp.float32),
        mesh=mesh,
        scratch_types=[
            pltpu.VMEM((PER_WORKER,), jnp.int32),
            pltpu.VMEM((PER_WORKER,), jnp.int32),
            pltpu.VMEM((3, CHUNK, PAIR), jnp.float32),
            pltpu.VMEM((3, CHUNK * EMB), jnp.float32),
            pltpu.SemaphoreType.DMA((3,)),
            pltpu.SemaphoreType.DMA((3,)),
        ],
        compiler_params=pltpu.CompilerParams(use_tc_tiling_on_sc=False),
    )
    return run(idx_flat, pair_id, pairs)


def kernel(unit_id_seqs, unit_embedding_table):
    idx_flat = unit_id_seqs.reshape(TOTAL).astype(jnp.int32)
    pair_id = idx_flat >> 1
    pairs = unit_embedding_table.reshape(VOCAB_PAIRS, PAIR)
    out = _embedding_gather(idx_flat, pair_id, pairs)
    return out.reshape(BATCH, SEQ, EMB)
